# baseline (speedup 1.0000x reference)
import functools

import jax
import jax.numpy as jnp
from jax import lax
from jax.experimental import pallas as pl
from jax.experimental.pallas import tpu as pltpu
from jax.experimental.pallas import tpu_sc as plsc

NUM_EMB = 8192
EMB_DIM = 256
N_TOK = 8192
COMMIT = 0.25
EPS = 1e-05

TB = 512
CB = 512
N_TB = N_TOK // TB
N_CB = NUM_EMB // CB

SEG = (0, 4096, NUM_EMB)
N_SEG = len(SEG) - 1


def _argmin_body(zb_ref, z2_ref, ewt_ref, e2_ref, idx_ref, counts_ref):
    step = pl.program_id(0)

    @pl.when(step == 0)
    def _init():
        counts_ref[...] = jnp.zeros_like(counts_ref)

    zb = zb_ref[...]
    z2 = z2_ref[...]

    big = jnp.int32(2147483647)
    seg_val = [jnp.full((TB, 1), jnp.inf, jnp.float32) for _ in range(N_SEG)]
    seg_idx = [jnp.zeros((TB, 1), jnp.int32) for _ in range(N_SEG)]
    for j in range(N_CB):
        lo, hi = j * CB, (j + 1) * CB
        ewt_c = ewt_ref[:, lo:hi]
        p = jnp.dot(zb, ewt_c, preferred_element_type=jnp.float32)
        dist = (z2 - p) + e2_ref[0:1, lo:hi]
        ids = lax.broadcasted_iota(jnp.int32, (TB, CB), 1) + lo
        for c in range(N_SEG):
            olo, ohi = max(lo, SEG[c]), min(hi, SEG[c + 1])
            if olo >= ohi:
                continue
            if olo == lo and ohi == hi:
                dm = dist
            else:
                inr = (ids >= olo) & (ids < ohi)
                dm = jnp.where(inr, dist, jnp.inf)
            vmin = jnp.min(dm, axis=1, keepdims=True)
            amin = jnp.min(jnp.where(dm == vmin, ids, big),
                           axis=1, keepdims=True)
            better = vmin < seg_val[c]
            seg_val[c] = jnp.where(better, vmin, seg_val[c])
            seg_idx[c] = jnp.where(better, amin, seg_idx[c])

    acc_v = seg_val[0].astype(jnp.bfloat16).astype(jnp.float32)
    acc_i = seg_idx[0]
    for c in range(1, N_SEG):
        take = seg_val[c] < acc_v
        acc_i = jnp.where(take, seg_idx[c], acc_i)
        acc_v = jnp.where(
            take, seg_val[c].astype(jnp.bfloat16).astype(jnp.float32), acc_v)
    best_idx = acc_i

    idx_ref[...] = best_idx

    for j in range(N_CB):
        ids = lax.broadcasted_iota(jnp.int32, (TB, CB), 1) + j * CB
        eq = (best_idx == ids).astype(jnp.float32)
        counts_ref[0:1, j * CB:(j + 1) * CB] += jnp.sum(eq, axis=0, keepdims=True)


def _argmin_call(zb_flat, z2_flat, ewt, e2):
    return pl.pallas_call(
        _argmin_body,
        grid=(N_TB,),
        in_specs=[
            pl.BlockSpec((TB, EMB_DIM), lambda i: (i, 0)),
            pl.BlockSpec((TB, 1), lambda i: (i, 0)),
            pl.BlockSpec((EMB_DIM, NUM_EMB), lambda i: (0, 0)),
            pl.BlockSpec((1, NUM_EMB), lambda i: (0, 0)),
        ],
        out_specs=[
            pl.BlockSpec((TB, 1), lambda i: (i, 0)),
            pl.BlockSpec((1, NUM_EMB), lambda i: (0, 0)),
        ],
        out_shape=[
            jax.ShapeDtypeStruct((N_TOK, 1), jnp.int32),
            jax.ShapeDtypeStruct((1, NUM_EMB), jnp.float32),
        ],
        compiler_params=pltpu.CompilerParams(
            dimension_semantics=("arbitrary",)),
    )(zb_flat, z2_flat, ewt, e2)


def _make_gather():
    info = plsc.get_sparse_core_info()
    nw = info.num_cores * info.num_subcores
    bpw = N_TOK // nw
    mesh = plsc.VectorSubcoreMesh(core_axis_name="c", subcore_axis_name="s")

    @functools.partial(
        pl.kernel, mesh=mesh,
        out_type=jax.ShapeDtypeStruct((N_TOK, EMB_DIM), jnp.float32),
        scratch_types=[
            pltpu.VMEM((bpw,), jnp.int32),
            pltpu.VMEM((bpw, EMB_DIM), jnp.float32),
            pltpu.SemaphoreType.DMA,
        ],
    )
    def gather_k(table_hbm, idx_hbm, out_hbm, idx_v, rows_v, sem):
        nc = info.num_cores
        wid = lax.axis_index("s") * nc + lax.axis_index("c")
        base = wid * bpw
        pltpu.sync_copy(idx_hbm.at[pl.ds(base, bpw)], idx_v)
        pltpu.async_copy(table_hbm.at[idx_v], rows_v, sem).wait()
        pltpu.sync_copy(rows_v, out_hbm.at[pl.ds(base, bpw)])

    return gather_k


def _gather_zq(ew, idx_flat):
    return _make_gather()(ew, idx_flat)


FB = EMB_DIM
N_FB = (N_TOK * EMB_DIM) // (FB * 1024)


def _finish_body(ze_ref, std_ref, zq_ref, counts_ref, zqst_ref, loss_ref,
                 perp_ref):
    step = pl.program_id(0)

    @pl.when(step == 0)
    def _init():
        loss_ref[...] = jnp.zeros((1, 1), jnp.float32)

    z = ze_ref[...] / std_ref[...]
    zq = zq_ref[...]
    d = z - zq
    loss_ref[...] += jnp.sum(d * d, axis=(0, 1), keepdims=True)
    zqst_ref[...] = z + (zq - z)

    @pl.when(step == N_FB - 1)
    def _fini():
        m = loss_ref[...] / (N_TOK * EMB_DIM)
        loss_ref[...] = COMMIT * m + m
        p = counts_ref[...] / N_TOK
        ent = jnp.sum(p * jnp.log(jnp.maximum(p, 1e-10)),
                      axis=(0, 1), keepdims=True)
        perp_ref[...] = jnp.exp(-ent)


def _finish_call(ze2d, std_col, zq2d, counts):
    return pl.pallas_call(
        _finish_body,
        grid=(N_FB,),
        in_specs=[
            pl.BlockSpec((FB, 1024), lambda i: (i, 0)),
            pl.BlockSpec((FB, 1), lambda i: (i, 0)),
            pl.BlockSpec((FB, 1024), lambda i: (i, 0)),
            pl.BlockSpec((1, NUM_EMB), lambda i: (0, 0)),
        ],
        out_specs=[
            pl.BlockSpec((FB, 1024), lambda i: (i, 0)),
            pl.BlockSpec((1, 1), lambda i: (0, 0)),
            pl.BlockSpec((1, 1), lambda i: (0, 0)),
        ],
        out_shape=[
            jax.ShapeDtypeStruct((N_TOK * EMB_DIM // 1024, 1024), jnp.float32),
            jax.ShapeDtypeStruct((1, 1), jnp.float32),
            jax.ShapeDtypeStruct((1, 1), jnp.float32),
        ],
        compiler_params=pltpu.CompilerParams(
            dimension_semantics=("arbitrary",)),
    )(ze2d, std_col, zq2d, counts)


def kernel(z_e, emb_w):
    b, c, h, w = z_e.shape
    z = z_e.astype(jnp.float32)
    std = jnp.std(z, axis=(0, 2, 3), keepdims=True, ddof=1)
    std = jnp.maximum(std, EPS)
    std = jax.lax.stop_gradient(std)
    z = z / std
    ew = emb_w.astype(jnp.float32)

    zb = (2.0 * z).astype(jnp.bfloat16)
    zb_flat = jnp.transpose(zb, (0, 2, 3, 1)).reshape(-1, c)
    z2_flat = jnp.sum(z * z, axis=1).reshape(-1, 1)
    e2 = jnp.sum(ew * ew, axis=1).reshape(1, -1)

    idx2, counts = _argmin_call(zb_flat, z2_flat, ew.T, e2)
    idx_flat = idx2.reshape(-1)
    zq_flat = _gather_zq(ew, idx_flat)

    zq_orig = zq_flat.reshape(b, h, w, c).transpose(0, 3, 1, 2)
    ze2d = z_e.reshape(b * c, h * w)
    zq2d = zq_orig.reshape(b * c, h * w)
    std_col = jnp.broadcast_to(std.reshape(1, c), (b, c)).reshape(b * c, 1)

    zqst2d, loss, perp = _finish_call(ze2d, std_col, zq2d, counts)

    z_q_st = zqst2d.reshape(b, c, h, w)
    vq_loss = loss.reshape(())
    perplexity = perp.reshape(())
    indices = idx_flat.reshape(b, h, w)
    return (z_q_st, vq_loss, perplexity, indices)


# speedup vs baseline: 1.4270x; 1.4270x over previous
import functools

import jax
import jax.numpy as jnp
from jax import lax
from jax.experimental import pallas as pl
from jax.experimental.pallas import tpu as pltpu
from jax.experimental.pallas import tpu_sc as plsc

NUM_EMB = 8192
EMB_DIM = 256
N_TOK = 8192
COMMIT = 0.25
EPS = 1e-05

TB = 512
CB = 512
N_TB = N_TOK // TB
N_CB = NUM_EMB // CB

SEG = (0, 4096, NUM_EMB)
N_SEG = len(SEG) - 1


def _argmin_body(z_ref, ewt_ref, idx_ref, counts_ref, e2_ref):
    step = pl.program_id(0)

    @pl.when(step == 0)
    def _init():
        ew = ewt_ref[...]
        e2_ref[...] = jnp.sum(ew * ew, axis=0, keepdims=True)
        counts_ref[...] = jnp.zeros_like(counts_ref)

    z = z_ref[...]
    z2 = jnp.sum(z * z, axis=1, keepdims=True)
    zb = (2.0 * z).astype(jnp.bfloat16)

    big = jnp.int32(2147483647)
    seg_val = [jnp.full((TB, 1), jnp.inf, jnp.float32) for _ in range(N_SEG)]
    seg_idx = [jnp.zeros((TB, 1), jnp.int32) for _ in range(N_SEG)]
    for j in range(N_CB):
        lo, hi = j * CB, (j + 1) * CB
        ewt_c = ewt_ref[:, lo:hi]
        p = jnp.dot(zb, ewt_c, preferred_element_type=jnp.float32)
        dist = (z2 - p) + e2_ref[0:1, lo:hi]
        ids = lax.broadcasted_iota(jnp.int32, (TB, CB), 1) + lo
        for c in range(N_SEG):
            olo, ohi = max(lo, SEG[c]), min(hi, SEG[c + 1])
            if olo >= ohi:
                continue
            if olo == lo and ohi == hi:
                dm = dist
            else:
                inr = (ids >= olo) & (ids < ohi)
                dm = jnp.where(inr, dist, jnp.inf)
            vmin = jnp.min(dm, axis=1, keepdims=True)
            amin = jnp.min(jnp.where(dm == vmin, ids, big),
                           axis=1, keepdims=True)
            better = vmin < seg_val[c]
            seg_val[c] = jnp.where(better, vmin, seg_val[c])
            seg_idx[c] = jnp.where(better, amin, seg_idx[c])

    acc_v = seg_val[0].astype(jnp.bfloat16).astype(jnp.float32)
    acc_i = seg_idx[0]
    for c in range(1, N_SEG):
        take = seg_val[c] < acc_v
        acc_i = jnp.where(take, seg_idx[c], acc_i)
        acc_v = jnp.where(
            take, seg_val[c].astype(jnp.bfloat16).astype(jnp.float32), acc_v)
    best_idx = acc_i

    idx_ref[...] = best_idx

    for j in range(N_CB):
        ids = lax.broadcasted_iota(jnp.int32, (TB, CB), 1) + j * CB
        eq = (best_idx == ids).astype(jnp.float32)
        counts_ref[0:1, j * CB:(j + 1) * CB] += jnp.sum(eq, axis=0, keepdims=True)


def _argmin_call(z_flat, ewt):
    return pl.pallas_call(
        _argmin_body,
        grid=(N_TB,),
        in_specs=[
            pl.BlockSpec((TB, EMB_DIM), lambda i: (i, 0)),
            pl.BlockSpec((EMB_DIM, NUM_EMB), lambda i: (0, 0)),
        ],
        out_specs=[
            pl.BlockSpec((TB, 1), lambda i: (i, 0)),
            pl.BlockSpec((1, NUM_EMB), lambda i: (0, 0)),
        ],
        out_shape=[
            jax.ShapeDtypeStruct((N_TOK, 1), jnp.int32),
            jax.ShapeDtypeStruct((1, NUM_EMB), jnp.float32),
        ],
        scratch_shapes=[pltpu.VMEM((1, NUM_EMB), jnp.float32)],
        compiler_params=pltpu.CompilerParams(
            dimension_semantics=("arbitrary",)),
    )(z_flat, ewt)


def _make_gather():
    info = plsc.get_sparse_core_info()
    nw = info.num_cores * info.num_subcores
    bpw = N_TOK // nw
    mesh = plsc.VectorSubcoreMesh(core_axis_name="c", subcore_axis_name="s")

    @functools.partial(
        pl.kernel, mesh=mesh,
        out_type=jax.ShapeDtypeStruct((N_TOK, EMB_DIM), jnp.float32),
        scratch_types=[
            pltpu.VMEM((bpw,), jnp.int32),
            pltpu.VMEM((bpw, EMB_DIM), jnp.float32),
            pltpu.SemaphoreType.DMA,
        ],
    )
    def gather_k(table_hbm, idx_hbm, out_hbm, idx_v, rows_v, sem):
        nc = info.num_cores
        wid = lax.axis_index("s") * nc + lax.axis_index("c")
        base = wid * bpw
        pltpu.sync_copy(idx_hbm.at[pl.ds(base, bpw)], idx_v)
        pltpu.async_copy(table_hbm.at[idx_v], rows_v, sem).wait()
        pltpu.sync_copy(rows_v, out_hbm.at[pl.ds(base, bpw)])

    return gather_k


def _gather_zq(ew, idx_flat):
    return _make_gather()(ew, idx_flat)


def _finish_body(z_ref, zq_ref, counts_ref, zqst_ref, loss_ref, perp_ref):
    step = pl.program_id(0)

    @pl.when(step == 0)
    def _init():
        loss_ref[...] = jnp.zeros((1, 1), jnp.float32)

    z = z_ref[...]
    zq = zq_ref[...]
    d = z - zq
    loss_ref[...] += jnp.sum(d * d, axis=(0, 1), keepdims=True)
    zqst_ref[...] = z + (zq - z)

    @pl.when(step == N_TB - 1)
    def _fini():
        m = loss_ref[...] / (N_TOK * EMB_DIM)
        loss_ref[...] = COMMIT * m + m
        p = counts_ref[...] / N_TOK
        ent = jnp.sum(p * jnp.log(jnp.maximum(p, 1e-10)),
                      axis=(0, 1), keepdims=True)
        perp_ref[...] = jnp.exp(-ent)


def _finish_call(z_flat, zq_flat, counts):
    return pl.pallas_call(
        _finish_body,
        grid=(N_TB,),
        in_specs=[
            pl.BlockSpec((TB, EMB_DIM), lambda i: (i, 0)),
            pl.BlockSpec((TB, EMB_DIM), lambda i: (i, 0)),
            pl.BlockSpec((1, NUM_EMB), lambda i: (0, 0)),
        ],
        out_specs=[
            pl.BlockSpec((TB, EMB_DIM), lambda i: (i, 0)),
            pl.BlockSpec((1, 1), lambda i: (0, 0)),
            pl.BlockSpec((1, 1), lambda i: (0, 0)),
        ],
        out_shape=[
            jax.ShapeDtypeStruct((N_TOK, EMB_DIM), jnp.float32),
            jax.ShapeDtypeStruct((1, 1), jnp.float32),
            jax.ShapeDtypeStruct((1, 1), jnp.float32),
        ],
        compiler_params=pltpu.CompilerParams(
            dimension_semantics=("arbitrary",)),
    )(z_flat, zq_flat, counts)


def kernel(z_e, emb_w):
    b, c, h, w = z_e.shape
    z = z_e.astype(jnp.float32)
    std = jnp.std(z, axis=(0, 2, 3), keepdims=True, ddof=1)
    std = jnp.maximum(std, EPS)
    std = jax.lax.stop_gradient(std)
    z = z / std
    z_flat = jnp.transpose(z, (0, 2, 3, 1)).reshape(-1, c)
    ew = emb_w.astype(jnp.float32)
    ewt = ew.T

    idx2, counts = _argmin_call(z_flat, ewt)
    idx_flat = idx2.reshape(-1)
    zq_flat = _gather_zq(ew, idx_flat)
    zqst_flat, loss, perp = _finish_call(z_flat, zq_flat, counts)

    z_q_st = zqst_flat.reshape(b, h, w, c).transpose(0, 3, 1, 2)
    vq_loss = loss.reshape(())
    perplexity = perp.reshape(())
    indices = idx_flat.reshape(b, h, w)
    return (z_q_st, vq_loss, perplexity, indices)


# speedup vs baseline: 1.5889x; 1.1135x over previous
import functools

import jax
import jax.numpy as jnp
from jax import lax
from jax.experimental import pallas as pl
from jax.experimental.pallas import tpu as pltpu
from jax.experimental.pallas import tpu_sc as plsc

NUM_EMB = 8192
EMB_DIM = 256
N_TOK = 8192
COMMIT = 0.25
EPS = 1e-05

TB = 512
CB = 2048
N_TB = N_TOK // TB
N_CB = NUM_EMB // CB

SEG = (0, 4096, NUM_EMB)
N_SEG = len(SEG) - 1


def _argmin_body(z_ref, ewt_ref, idx_ref, counts_ref, e2_ref):
    step = pl.program_id(0)

    @pl.when(step == 0)
    def _init():
        ew = ewt_ref[...]
        e2_ref[...] = jnp.sum(ew * ew, axis=0, keepdims=True)
        counts_ref[...] = jnp.zeros_like(counts_ref)

    z = z_ref[...]
    z2 = jnp.sum(z * z, axis=1, keepdims=True)
    zb = (2.0 * z).astype(jnp.bfloat16)

    big = jnp.int32(2147483647)
    seg_val = [jnp.full((TB, 1), jnp.inf, jnp.float32) for _ in range(N_SEG)]
    seg_idx = [jnp.zeros((TB, 1), jnp.int32) for _ in range(N_SEG)]
    for j in range(N_CB):
        lo, hi = j * CB, (j + 1) * CB
        ewt_c = ewt_ref[:, lo:hi]
        p = jnp.dot(zb, ewt_c, preferred_element_type=jnp.float32)
        dist = (z2 - p) + e2_ref[0:1, lo:hi]
        ids = lax.broadcasted_iota(jnp.int32, (TB, CB), 1) + lo
        for c in range(N_SEG):
            olo, ohi = max(lo, SEG[c]), min(hi, SEG[c + 1])
            if olo >= ohi:
                continue
            if olo == lo and ohi == hi:
                dm = dist
            else:
                inr = (ids >= olo) & (ids < ohi)
                dm = jnp.where(inr, dist, jnp.inf)
            vmin = jnp.min(dm, axis=1, keepdims=True)
            amin = jnp.min(jnp.where(dm == vmin, ids, big),
                           axis=1, keepdims=True)
            better = vmin < seg_val[c]
            seg_val[c] = jnp.where(better, vmin, seg_val[c])
            seg_idx[c] = jnp.where(better, amin, seg_idx[c])

    acc_v = seg_val[0].astype(jnp.bfloat16).astype(jnp.float32)
    acc_i = seg_idx[0]
    for c in range(1, N_SEG):
        take = seg_val[c] < acc_v
        acc_i = jnp.where(take, seg_idx[c], acc_i)
        acc_v = jnp.where(
            take, seg_val[c].astype(jnp.bfloat16).astype(jnp.float32), acc_v)
    best_idx = acc_i

    idx_ref[...] = best_idx

    for j in range(N_CB):
        ids = lax.broadcasted_iota(jnp.int32, (TB, CB), 1) + j * CB
        eq = (best_idx == ids).astype(jnp.float32)
        counts_ref[0:1, j * CB:(j + 1) * CB] += jnp.sum(eq, axis=0, keepdims=True)


def _argmin_call(z_flat, ewt):
    return pl.pallas_call(
        _argmin_body,
        grid=(N_TB,),
        in_specs=[
            pl.BlockSpec((TB, EMB_DIM), lambda i: (i, 0)),
            pl.BlockSpec((EMB_DIM, NUM_EMB), lambda i: (0, 0)),
        ],
        out_specs=[
            pl.BlockSpec((TB, 1), lambda i: (i, 0)),
            pl.BlockSpec((1, NUM_EMB), lambda i: (0, 0)),
        ],
        out_shape=[
            jax.ShapeDtypeStruct((N_TOK, 1), jnp.int32),
            jax.ShapeDtypeStruct((1, NUM_EMB), jnp.float32),
        ],
        scratch_shapes=[pltpu.VMEM((1, NUM_EMB), jnp.float32)],
        compiler_params=pltpu.CompilerParams(
            dimension_semantics=("arbitrary",)),
    )(z_flat, ewt)


def _make_gather():
    info = plsc.get_sparse_core_info()
    nw = info.num_cores * info.num_subcores
    bpw = N_TOK // nw
    mesh = plsc.VectorSubcoreMesh(core_axis_name="c", subcore_axis_name="s")

    @functools.partial(
        pl.kernel, mesh=mesh,
        out_type=jax.ShapeDtypeStruct((N_TOK, EMB_DIM), jnp.float32),
        scratch_types=[
            pltpu.VMEM((bpw,), jnp.int32),
            pltpu.VMEM((bpw, EMB_DIM), jnp.float32),
            pltpu.SemaphoreType.DMA,
        ],
    )
    def gather_k(table_hbm, idx_hbm, out_hbm, idx_v, rows_v, sem):
        nc = info.num_cores
        wid = lax.axis_index("s") * nc + lax.axis_index("c")
        base = wid * bpw
        pltpu.sync_copy(idx_hbm.at[pl.ds(base, bpw)], idx_v)
        pltpu.async_copy(table_hbm.at[idx_v], rows_v, sem).wait()
        pltpu.sync_copy(rows_v, out_hbm.at[pl.ds(base, bpw)])

    return gather_k


def _gather_zq(ew, idx_flat):
    return _make_gather()(ew, idx_flat)


def _finish_body(z_ref, zq_ref, counts_ref, zqst_ref, loss_ref, perp_ref):
    step = pl.program_id(0)

    @pl.when(step == 0)
    def _init():
        loss_ref[...] = jnp.zeros((1, 1), jnp.float32)

    z = z_ref[...]
    zq = zq_ref[...]
    d = z - zq
    loss_ref[...] += jnp.sum(d * d, axis=(0, 1), keepdims=True)
    zqst_ref[...] = z + (zq - z)

    @pl.when(step == N_TB - 1)
    def _fini():
        m = loss_ref[...] / (N_TOK * EMB_DIM)
        loss_ref[...] = COMMIT * m + m
        p = counts_ref[...] / N_TOK
        ent = jnp.sum(p * jnp.log(jnp.maximum(p, 1e-10)),
                      axis=(0, 1), keepdims=True)
        perp_ref[...] = jnp.exp(-ent)


def _finish_call(z_flat, zq_flat, counts):
    return pl.pallas_call(
        _finish_body,
        grid=(N_TB,),
        in_specs=[
            pl.BlockSpec((TB, EMB_DIM), lambda i: (i, 0)),
            pl.BlockSpec((TB, EMB_DIM), lambda i: (i, 0)),
            pl.BlockSpec((1, NUM_EMB), lambda i: (0, 0)),
        ],
        out_specs=[
            pl.BlockSpec((TB, EMB_DIM), lambda i: (i, 0)),
            pl.BlockSpec((1, 1), lambda i: (0, 0)),
            pl.BlockSpec((1, 1), lambda i: (0, 0)),
        ],
        out_shape=[
            jax.ShapeDtypeStruct((N_TOK, EMB_DIM), jnp.float32),
            jax.ShapeDtypeStruct((1, 1), jnp.float32),
            jax.ShapeDtypeStruct((1, 1), jnp.float32),
        ],
        compiler_params=pltpu.CompilerParams(
            dimension_semantics=("arbitrary",)),
    )(z_flat, zq_flat, counts)


def kernel(z_e, emb_w):
    b, c, h, w = z_e.shape
    z = z_e.astype(jnp.float32)
    std = jnp.std(z, axis=(0, 2, 3), keepdims=True, ddof=1)
    std = jnp.maximum(std, EPS)
    std = jax.lax.stop_gradient(std)
    z = z / std
    z_flat = jnp.transpose(z, (0, 2, 3, 1)).reshape(-1, c)
    ew = emb_w.astype(jnp.float32)
    ewt = ew.T

    idx2, counts = _argmin_call(z_flat, ewt)
    idx_flat = idx2.reshape(-1)
    zq_flat = _gather_zq(ew, idx_flat)
    zqst_flat, loss, perp = _finish_call(z_flat, zq_flat, counts)

    z_q_st = zqst_flat.reshape(b, h, w, c).transpose(0, 3, 1, 2)
    vq_loss = loss.reshape(())
    perplexity = perp.reshape(())
    indices = idx_flat.reshape(b, h, w)
    return (z_q_st, vq_loss, perplexity, indices)


# speedup vs baseline: 1.6072x; 1.0115x over previous
import functools

import jax
import jax.numpy as jnp
from jax import lax
from jax.experimental import pallas as pl
from jax.experimental.pallas import tpu as pltpu
from jax.experimental.pallas import tpu_sc as plsc

NUM_EMB = 8192
EMB_DIM = 256
N_TOK = 8192
COMMIT = 0.25
EPS = 1e-05

TB = 512
CB = 4096
N_TB = N_TOK // TB
N_CB = NUM_EMB // CB

SEG = (0, 4096, NUM_EMB)
N_SEG = len(SEG) - 1


def _argmin_body(z_ref, ewt_ref, idx_ref, counts_ref, e2_ref):
    step = pl.program_id(0)

    @pl.when(step == 0)
    def _init():
        ew = ewt_ref[...]
        e2_ref[...] = jnp.sum(ew * ew, axis=0, keepdims=True)
        counts_ref[...] = jnp.zeros_like(counts_ref)

    z = z_ref[...]
    z2 = jnp.sum(z * z, axis=1, keepdims=True)
    zb = (2.0 * z).astype(jnp.bfloat16)

    big = jnp.int32(2147483647)
    seg_val = [jnp.full((TB, 1), jnp.inf, jnp.float32) for _ in range(N_SEG)]
    seg_idx = [jnp.zeros((TB, 1), jnp.int32) for _ in range(N_SEG)]
    for j in range(N_CB):
        lo, hi = j * CB, (j + 1) * CB
        ewt_c = ewt_ref[:, lo:hi]
        p = jnp.dot(zb, ewt_c, preferred_element_type=jnp.float32)
        dist = (z2 - p) + e2_ref[0:1, lo:hi]
        ids = lax.broadcasted_iota(jnp.int32, (TB, CB), 1) + lo
        for c in range(N_SEG):
            olo, ohi = max(lo, SEG[c]), min(hi, SEG[c + 1])
            if olo >= ohi:
                continue
            if olo == lo and ohi == hi:
                dm = dist
            else:
                inr = (ids >= olo) & (ids < ohi)
                dm = jnp.where(inr, dist, jnp.inf)
            vmin = jnp.min(dm, axis=1, keepdims=True)
            amin = jnp.min(jnp.where(dm == vmin, ids, big),
                           axis=1, keepdims=True)
            better = vmin < seg_val[c]
            seg_val[c] = jnp.where(better, vmin, seg_val[c])
            seg_idx[c] = jnp.where(better, amin, seg_idx[c])

    acc_v = seg_val[0].astype(jnp.bfloat16).astype(jnp.float32)
    acc_i = seg_idx[0]
    for c in range(1, N_SEG):
        take = seg_val[c] < acc_v
        acc_i = jnp.where(take, seg_idx[c], acc_i)
        acc_v = jnp.where(
            take, seg_val[c].astype(jnp.bfloat16).astype(jnp.float32), acc_v)
    best_idx = acc_i

    idx_ref[...] = best_idx

    for j in range(N_CB):
        ids = lax.broadcasted_iota(jnp.int32, (TB, CB), 1) + j * CB
        eq = (best_idx == ids).astype(jnp.float32)
        counts_ref[0:1, j * CB:(j + 1) * CB] += jnp.sum(eq, axis=0, keepdims=True)


def _argmin_call(z_flat, ewt):
    return pl.pallas_call(
        _argmin_body,
        grid=(N_TB,),
        in_specs=[
            pl.BlockSpec((TB, EMB_DIM), lambda i: (i, 0)),
            pl.BlockSpec((EMB_DIM, NUM_EMB), lambda i: (0, 0)),
        ],
        out_specs=[
            pl.BlockSpec((TB, 1), lambda i: (i, 0)),
            pl.BlockSpec((1, NUM_EMB), lambda i: (0, 0)),
        ],
        out_shape=[
            jax.ShapeDtypeStruct((N_TOK, 1), jnp.int32),
            jax.ShapeDtypeStruct((1, NUM_EMB), jnp.float32),
        ],
        scratch_shapes=[pltpu.VMEM((1, NUM_EMB), jnp.float32)],
        compiler_params=pltpu.CompilerParams(
            dimension_semantics=("arbitrary",)),
    )(z_flat, ewt)


def _make_gather():
    info = plsc.get_sparse_core_info()
    nw = info.num_cores * info.num_subcores
    bpw = N_TOK // nw
    mesh = plsc.VectorSubcoreMesh(core_axis_name="c", subcore_axis_name="s")

    @functools.partial(
        pl.kernel, mesh=mesh,
        out_type=jax.ShapeDtypeStruct((N_TOK, EMB_DIM), jnp.float32),
        scratch_types=[
            pltpu.VMEM((bpw,), jnp.int32),
            pltpu.VMEM((bpw, EMB_DIM), jnp.float32),
            pltpu.SemaphoreType.DMA,
        ],
    )
    def gather_k(table_hbm, idx_hbm, out_hbm, idx_v, rows_v, sem):
        nc = info.num_cores
        wid = lax.axis_index("s") * nc + lax.axis_index("c")
        base = wid * bpw
        pltpu.sync_copy(idx_hbm.at[pl.ds(base, bpw)], idx_v)
        pltpu.async_copy(table_hbm.at[idx_v], rows_v, sem).wait()
        pltpu.sync_copy(rows_v, out_hbm.at[pl.ds(base, bpw)])

    return gather_k


def _gather_zq(ew, idx_flat):
    return _make_gather()(ew, idx_flat)


def _finish_body(z_ref, zq_ref, counts_ref, zqst_ref, loss_ref, perp_ref):
    step = pl.program_id(0)

    @pl.when(step == 0)
    def _init():
        loss_ref[...] = jnp.zeros((1, 1), jnp.float32)

    z = z_ref[...]
    zq = zq_ref[...]
    d = z - zq
    loss_ref[...] += jnp.sum(d * d, axis=(0, 1), keepdims=True)
    zqst_ref[...] = z + (zq - z)

    @pl.when(step == N_TB - 1)
    def _fini():
        m = loss_ref[...] / (N_TOK * EMB_DIM)
        loss_ref[...] = COMMIT * m + m
        p = counts_ref[...] / N_TOK
        ent = jnp.sum(p * jnp.log(jnp.maximum(p, 1e-10)),
                      axis=(0, 1), keepdims=True)
        perp_ref[...] = jnp.exp(-ent)


def _finish_call(z_flat, zq_flat, counts):
    return pl.pallas_call(
        _finish_body,
        grid=(N_TB,),
        in_specs=[
            pl.BlockSpec((TB, EMB_DIM), lambda i: (i, 0)),
            pl.BlockSpec((TB, EMB_DIM), lambda i: (i, 0)),
            pl.BlockSpec((1, NUM_EMB), lambda i: (0, 0)),
        ],
        out_specs=[
            pl.BlockSpec((TB, EMB_DIM), lambda i: (i, 0)),
            pl.BlockSpec((1, 1), lambda i: (0, 0)),
            pl.BlockSpec((1, 1), lambda i: (0, 0)),
        ],
        out_shape=[
            jax.ShapeDtypeStruct((N_TOK, EMB_DIM), jnp.float32),
            jax.ShapeDtypeStruct((1, 1), jnp.float32),
            jax.ShapeDtypeStruct((1, 1), jnp.float32),
        ],
        compiler_params=pltpu.CompilerParams(
            dimension_semantics=("arbitrary",)),
    )(z_flat, zq_flat, counts)


def kernel(z_e, emb_w):
    b, c, h, w = z_e.shape
    z = z_e.astype(jnp.float32)
    std = jnp.std(z, axis=(0, 2, 3), keepdims=True, ddof=1)
    std = jnp.maximum(std, EPS)
    std = jax.lax.stop_gradient(std)
    z = z / std
    z_flat = jnp.transpose(z, (0, 2, 3, 1)).reshape(-1, c)
    ew = emb_w.astype(jnp.float32)
    ewt = ew.T

    idx2, counts = _argmin_call(z_flat, ewt)
    idx_flat = idx2.reshape(-1)
    zq_flat = _gather_zq(ew, idx_flat)
    zqst_flat, loss, perp = _finish_call(z_flat, zq_flat, counts)

    z_q_st = zqst_flat.reshape(b, h, w, c).transpose(0, 3, 1, 2)
    vq_loss = loss.reshape(())
    perplexity = perp.reshape(())
    indices = idx_flat.reshape(b, h, w)
    return (z_q_st, vq_loss, perplexity, indices)


# speedup vs baseline: 1.6893x; 1.0511x over previous
import functools

import jax
import jax.numpy as jnp
from jax import lax
from jax.experimental import pallas as pl
from jax.experimental.pallas import tpu as pltpu
from jax.experimental.pallas import tpu_sc as plsc

NUM_EMB = 8192
EMB_DIM = 256
N_TOK = 8192
COMMIT = 0.25
EPS = 1e-05

TB = 1024
CB = 4096
N_TB = N_TOK // TB
N_CB = NUM_EMB // CB

SEG = (0, 4096, NUM_EMB)
N_SEG = len(SEG) - 1


def _argmin_body(z_ref, ewt_ref, idx_ref, counts_ref, e2_ref):
    step = pl.program_id(0)

    @pl.when(step == 0)
    def _init():
        ew = ewt_ref[...]
        e2_ref[...] = jnp.sum(ew * ew, axis=0, keepdims=True)
        counts_ref[...] = jnp.zeros_like(counts_ref)

    z = z_ref[...]
    z2 = jnp.sum(z * z, axis=1, keepdims=True)
    zb = (2.0 * z).astype(jnp.bfloat16)

    big = jnp.int32(2147483647)
    seg_val = [jnp.full((TB, 1), jnp.inf, jnp.float32) for _ in range(N_SEG)]
    seg_idx = [jnp.zeros((TB, 1), jnp.int32) for _ in range(N_SEG)]
    for j in range(N_CB):
        lo, hi = j * CB, (j + 1) * CB
        ewt_c = ewt_ref[:, lo:hi]
        p = jnp.dot(zb, ewt_c, preferred_element_type=jnp.float32)
        dist = (z2 - p) + e2_ref[0:1, lo:hi]
        ids = lax.broadcasted_iota(jnp.int32, (TB, CB), 1) + lo
        for c in range(N_SEG):
            olo, ohi = max(lo, SEG[c]), min(hi, SEG[c + 1])
            if olo >= ohi:
                continue
            if olo == lo and ohi == hi:
                dm = dist
            else:
                inr = (ids >= olo) & (ids < ohi)
                dm = jnp.where(inr, dist, jnp.inf)
            vmin = jnp.min(dm, axis=1, keepdims=True)
            amin = jnp.min(jnp.where(dm == vmin, ids, big),
                           axis=1, keepdims=True)
            better = vmin < seg_val[c]
            seg_val[c] = jnp.where(better, vmin, seg_val[c])
            seg_idx[c] = jnp.where(better, amin, seg_idx[c])

    acc_v = seg_val[0].astype(jnp.bfloat16).astype(jnp.float32)
    acc_i = seg_idx[0]
    for c in range(1, N_SEG):
        take = seg_val[c] < acc_v
        acc_i = jnp.where(take, seg_idx[c], acc_i)
        acc_v = jnp.where(
            take, seg_val[c].astype(jnp.bfloat16).astype(jnp.float32), acc_v)
    best_idx = acc_i

    idx_ref[...] = best_idx

    for j in range(N_CB):
        ids = lax.broadcasted_iota(jnp.int32, (TB, CB), 1) + j * CB
        eq = (best_idx == ids).astype(jnp.float32)
        counts_ref[0:1, j * CB:(j + 1) * CB] += jnp.sum(eq, axis=0, keepdims=True)


def _argmin_call(z_flat, ewt):
    return pl.pallas_call(
        _argmin_body,
        grid=(N_TB,),
        in_specs=[
            pl.BlockSpec((TB, EMB_DIM), lambda i: (i, 0)),
            pl.BlockSpec((EMB_DIM, NUM_EMB), lambda i: (0, 0)),
        ],
        out_specs=[
            pl.BlockSpec((TB, 1), lambda i: (i, 0)),
            pl.BlockSpec((1, NUM_EMB), lambda i: (0, 0)),
        ],
        out_shape=[
            jax.ShapeDtypeStruct((N_TOK, 1), jnp.int32),
            jax.ShapeDtypeStruct((1, NUM_EMB), jnp.float32),
        ],
        scratch_shapes=[pltpu.VMEM((1, NUM_EMB), jnp.float32)],
        compiler_params=pltpu.CompilerParams(
            dimension_semantics=("arbitrary",)),
    )(z_flat, ewt)


def _make_gather():
    info = plsc.get_sparse_core_info()
    nw = info.num_cores * info.num_subcores
    bpw = N_TOK // nw
    mesh = plsc.VectorSubcoreMesh(core_axis_name="c", subcore_axis_name="s")

    @functools.partial(
        pl.kernel, mesh=mesh,
        out_type=jax.ShapeDtypeStruct((N_TOK, EMB_DIM), jnp.float32),
        scratch_types=[
            pltpu.VMEM((bpw,), jnp.int32),
            pltpu.VMEM((bpw, EMB_DIM), jnp.float32),
            pltpu.SemaphoreType.DMA,
        ],
    )
    def gather_k(table_hbm, idx_hbm, out_hbm, idx_v, rows_v, sem):
        nc = info.num_cores
        wid = lax.axis_index("s") * nc + lax.axis_index("c")
        base = wid * bpw
        pltpu.sync_copy(idx_hbm.at[pl.ds(base, bpw)], idx_v)
        pltpu.async_copy(table_hbm.at[idx_v], rows_v, sem).wait()
        pltpu.sync_copy(rows_v, out_hbm.at[pl.ds(base, bpw)])

    return gather_k


def _gather_zq(ew, idx_flat):
    return _make_gather()(ew, idx_flat)


def _finish_body(z_ref, zq_ref, counts_ref, zqst_ref, loss_ref, perp_ref):
    step = pl.program_id(0)

    @pl.when(step == 0)
    def _init():
        loss_ref[...] = jnp.zeros((1, 1), jnp.float32)

    z = z_ref[...]
    zq = zq_ref[...]
    d = z - zq
    loss_ref[...] += jnp.sum(d * d, axis=(0, 1), keepdims=True)
    zqst_ref[...] = z + (zq - z)

    @pl.when(step == N_TB - 1)
    def _fini():
        m = loss_ref[...] / (N_TOK * EMB_DIM)
        loss_ref[...] = COMMIT * m + m
        p = counts_ref[...] / N_TOK
        ent = jnp.sum(p * jnp.log(jnp.maximum(p, 1e-10)),
                      axis=(0, 1), keepdims=True)
        perp_ref[...] = jnp.exp(-ent)


def _finish_call(z_flat, zq_flat, counts):
    return pl.pallas_call(
        _finish_body,
        grid=(N_TB,),
        in_specs=[
            pl.BlockSpec((TB, EMB_DIM), lambda i: (i, 0)),
            pl.BlockSpec((TB, EMB_DIM), lambda i: (i, 0)),
            pl.BlockSpec((1, NUM_EMB), lambda i: (0, 0)),
        ],
        out_specs=[
            pl.BlockSpec((TB, EMB_DIM), lambda i: (i, 0)),
            pl.BlockSpec((1, 1), lambda i: (0, 0)),
            pl.BlockSpec((1, 1), lambda i: (0, 0)),
        ],
        out_shape=[
            jax.ShapeDtypeStruct((N_TOK, EMB_DIM), jnp.float32),
            jax.ShapeDtypeStruct((1, 1), jnp.float32),
            jax.ShapeDtypeStruct((1, 1), jnp.float32),
        ],
        compiler_params=pltpu.CompilerParams(
            dimension_semantics=("arbitrary",)),
    )(z_flat, zq_flat, counts)


def kernel(z_e, emb_w):
    b, c, h, w = z_e.shape
    z = z_e.astype(jnp.float32)
    std = jnp.std(z, axis=(0, 2, 3), keepdims=True, ddof=1)
    std = jnp.maximum(std, EPS)
    std = jax.lax.stop_gradient(std)
    z = z / std
    z_flat = jnp.transpose(z, (0, 2, 3, 1)).reshape(-1, c)
    ew = emb_w.astype(jnp.float32)
    ewt = ew.T

    idx2, counts = _argmin_call(z_flat, ewt)
    idx_flat = idx2.reshape(-1)
    zq_flat = _gather_zq(ew, idx_flat)
    zqst_flat, loss, perp = _finish_call(z_flat, zq_flat, counts)

    z_q_st = zqst_flat.reshape(b, h, w, c).transpose(0, 3, 1, 2)
    vq_loss = loss.reshape(())
    perplexity = perp.reshape(())
    indices = idx_flat.reshape(b, h, w)
    return (z_q_st, vq_loss, perplexity, indices)
